# baseline (speedup 1.0000x reference)
import functools

import jax
import jax.numpy as jnp
from jax import lax
from jax.experimental import pallas as pl
from jax.experimental.pallas import tpu as pltpu
from jax.experimental.pallas import tpu_sc as plsc

_NUM_CORES = 2
_NUM_SUBCORES = 16
_NUM_WORKERS = _NUM_CORES * _NUM_SUBCORES

_CHUNK = 512
_NBUF = 2

_RETILE_W = 4096


@functools.lru_cache(maxsize=None)
def _make_retile(vocab: int, d: int):
  grid = -(-vocab // _RETILE_W)

  def retile_body(tT_ref, out_ref):
    x = tT_ref[...]
    xt = jax.lax.dot_general(x, jnp.eye(d, dtype=jnp.float32),
                             (((0,), (0,)), ((), ())))
    padded = jnp.concatenate(
        [xt, jnp.zeros((_RETILE_W, d), jnp.float32)], axis=1)
    out_ref[...] = jnp.reshape(padded, (_RETILE_W * 2 * d,))

  return pl.pallas_call(
      retile_body,
      grid=(grid,),
      in_specs=[pl.BlockSpec((d, _RETILE_W), lambda i: (0, i))],
      out_specs=pl.BlockSpec((_RETILE_W * 2 * d,), lambda i: (i,)),
      out_shape=jax.ShapeDtypeStruct((vocab * 2 * d,), jnp.float32),
  )


@functools.lru_cache(maxsize=None)
def _make_gather(n: int, vocab2: int, d: int):
  n_per_w = n // _NUM_WORKERS
  n_chunks = n_per_w // _CHUNK
  n_groups = n_chunks // _NBUF
  assert n == n_per_w * _NUM_WORKERS
  assert n_per_w == n_chunks * _CHUNK
  assert n_chunks == n_groups * _NBUF
  mesh = plsc.VectorSubcoreMesh(
      core_axis_name="c", subcore_axis_name="s",
      num_cores=_NUM_CORES, num_subcores=_NUM_SUBCORES)

  @functools.partial(
      pl.kernel,
      out_type=jax.ShapeDtypeStruct((n, 2 * d), jnp.float32),
      mesh=mesh,
      compiler_params=pltpu.CompilerParams(use_tc_tiling_on_sc=False),
      scratch_types=[
          pltpu.VMEM((n_per_w,), jnp.int32),
          [pltpu.VMEM((_CHUNK, d), jnp.float32) for _ in range(_NBUF)],
          [pltpu.SemaphoreType.DMA for _ in range(_NBUF)],
          [pltpu.SemaphoreType.DMA for _ in range(_NBUF)],
      ],
  )
  def gather_kernel(idx_hbm, table_hbm, out_hbm, idx_all, rows, gsem, osem):
    wid = lax.axis_index("s") * _NUM_CORES + lax.axis_index("c")
    base = wid * n_per_w
    pltpu.sync_copy(idx_hbm.at[pl.ds(base, n_per_w)], idx_all)

    def fire_gather(chunk, b):
      pltpu.async_copy(
          table_hbm.at[idx_all.at[pl.ds(chunk * _CHUNK, _CHUNK)]],
          rows[b], gsem[b])

    def wait_gather(b):
      pltpu.make_async_copy(table_hbm.at[idx_all.at[pl.ds(0, _CHUNK)]],
                            rows[b], gsem[b]).wait()

    def fire_out(chunk, b):
      pltpu.async_copy(
          rows[b],
          out_hbm.at[pl.ds(base + chunk * _CHUNK, _CHUNK), pl.ds(0, d)],
          osem[b])

    def wait_out(b):
      pltpu.make_async_copy(rows[b],
                            out_hbm.at[pl.ds(base, _CHUNK), pl.ds(0, d)],
                            osem[b]).wait()

    for b in range(_NBUF):
      fire_gather(b, b)

    def group_body(g, carry):
      for b in range(_NBUF):
        wait_gather(b)
        fire_out(g * _NBUF + b, b)
      for b in range(_NBUF):
        wait_out(b)
        fire_gather((g + 1) * _NBUF + b, b)
      return carry

    lax.fori_loop(0, n_groups - 1, group_body, 0)

    for b in range(_NBUF):
      wait_gather(b)
      fire_out((n_groups - 1) * _NBUF + b, b)
    for b in range(_NBUF):
      wait_out(b)

  return gather_kernel


def kernel(input, table):
  b, s, _ = input.shape
  vocab, d = table.shape
  n = b * s
  table2 = jnp.reshape(_make_retile(vocab, d)(table.T), (2 * vocab, d))
  idx2 = jnp.reshape(input, (n,)).astype(jnp.int32) * 2
  out2 = _make_gather(n, 2 * vocab, d)(idx2, table2)
  return jnp.reshape(out2[:, :d], (b, s, d))


# speedup vs baseline: 1.0257x; 1.0257x over previous
import functools

import jax
import jax.numpy as jnp
from jax import lax
from jax.experimental import pallas as pl
from jax.experimental.pallas import tpu as pltpu
from jax.experimental.pallas import tpu_sc as plsc

_NUM_CORES = 2
_NUM_SUBCORES = 16
_NUM_WORKERS = _NUM_CORES * _NUM_SUBCORES

_CHUNK = 512
_NBUF = 2

_RETILE_W = 4096


@functools.lru_cache(maxsize=None)
def _make_retile(vocab: int, d: int):
  grid = -(-vocab // _RETILE_W)

  def retile_body(tT_ref, out_ref):
    x = tT_ref[...]
    xt = jnp.transpose(x)
    y = jnp.concatenate([xt[:_RETILE_W // 2], xt[_RETILE_W // 2:]], axis=1)
    out_ref[...] = jnp.reshape(y, (_RETILE_W * d,))

  return pl.pallas_call(
      retile_body,
      grid=(grid,),
      in_specs=[pl.BlockSpec((d, _RETILE_W), lambda i: (0, i))],
      out_specs=pl.BlockSpec((_RETILE_W * d,), lambda i: (i,)),
      out_shape=jax.ShapeDtypeStruct((grid * _RETILE_W * d,), jnp.float32),
  )


@functools.lru_cache(maxsize=None)
def _make_gather(n: int, vocab2: int, d: int):
  n_per_w = n // _NUM_WORKERS
  n_chunks = n_per_w // _CHUNK
  n_groups = n_chunks // _NBUF
  assert n == n_per_w * _NUM_WORKERS
  assert n_per_w == n_chunks * _CHUNK
  assert n_chunks == n_groups * _NBUF
  mesh = plsc.VectorSubcoreMesh(
      core_axis_name="c", subcore_axis_name="s",
      num_cores=_NUM_CORES, num_subcores=_NUM_SUBCORES)

  @functools.partial(
      pl.kernel,
      out_type=jax.ShapeDtypeStruct((n, 2 * d), jnp.float32),
      mesh=mesh,
      compiler_params=pltpu.CompilerParams(use_tc_tiling_on_sc=False),
      scratch_types=[
          pltpu.VMEM((n_per_w,), jnp.int32),
          [pltpu.VMEM((_CHUNK, d), jnp.float32) for _ in range(_NBUF)],
          [pltpu.SemaphoreType.DMA for _ in range(_NBUF)],
          [pltpu.SemaphoreType.DMA for _ in range(_NBUF)],
      ],
  )
  def gather_kernel(idx_hbm, table_hbm, out_hbm, idx_all, rows, gsem, osem):
    wid = lax.axis_index("s") * _NUM_CORES + lax.axis_index("c")
    base = wid * n_per_w
    pltpu.sync_copy(idx_hbm.at[pl.ds(base, n_per_w)], idx_all)

    def fire_gather(chunk, b):
      pltpu.async_copy(
          table_hbm.at[idx_all.at[pl.ds(chunk * _CHUNK, _CHUNK)]],
          rows[b], gsem[b])

    def wait_gather(b):
      pltpu.make_async_copy(table_hbm.at[idx_all.at[pl.ds(0, _CHUNK)]],
                            rows[b], gsem[b]).wait()

    def fire_out(chunk, b):
      pltpu.async_copy(
          rows[b],
          out_hbm.at[pl.ds(base + chunk * _CHUNK, _CHUNK), pl.ds(0, d)],
          osem[b])

    def wait_out(b):
      pltpu.make_async_copy(rows[b],
                            out_hbm.at[pl.ds(base, _CHUNK), pl.ds(0, d)],
                            osem[b]).wait()

    for b in range(_NBUF):
      fire_gather(b, b)

    def group_body(g, carry):
      for b in range(_NBUF):
        wait_gather(b)
        fire_out(g * _NBUF + b, b)
      for b in range(_NBUF):
        wait_out(b)
        fire_gather((g + 1) * _NBUF + b, b)
      return carry

    lax.fori_loop(0, n_groups - 1, group_body, 0)

    for b in range(_NBUF):
      wait_gather(b)
      fire_out((n_groups - 1) * _NBUF + b, b)
    for b in range(_NBUF):
      wait_out(b)

  return gather_kernel


def kernel(input, table):
  b, s, _ = input.shape
  vocab, d = table.shape
  n = b * s
  grid = -(-vocab // _RETILE_W)
  table_rows = jnp.reshape(_make_retile(vocab, d)(table.T),
                           (grid * _RETILE_W, d))
  v = jnp.reshape(input, (n,)).astype(jnp.int32)
  q = v & (_RETILE_W - 1)
  gidx = ((v >> 12) << 12) + ((q & (_RETILE_W // 2 - 1)) << 1) + (q >> 11)
  out2 = _make_gather(n, grid * _RETILE_W, d)(gidx, table_rows)
  return jnp.reshape(out2[:, :d], (b, s, d))
